# Initial kernel scaffold; baseline (speedup 1.0000x reference)
import jax
import jax.numpy as jnp
from jax.experimental import pallas as pl


def kernel(x, edge_index, Wl1, bl1, Wr1, Wl2, bl2, Wr2, Wl3, bl3, Wr3, Wl4, bl4, Wr4, Wl5, bl5, Wr5, Wl6, bl6, Wr6):
    raise NotImplementedError("write your pallas kernel here")


# speedup vs baseline: 2.3001x; 2.3001x over previous
import functools

import jax
import jax.numpy as jnp
from jax import lax
from jax.experimental import pallas as pl
from jax.experimental.pallas import tpu as pltpu
from jax.experimental.pallas import tpu_sc as plsc

N = 10000
E = 320000
D = 128

NC = 2
NS = 16
NW = NC * NS
NPT = 320
NPAD = NW * NPT
K = 128


def _worker_id():
    return lax.axis_index("s") * NC + lax.axis_index("c")


def _seg_max_body(src_hbm, dst_hbm, est_hbm, h_hbm, out_hbm,
                  ebuf, idx_v, dstb, rows_v, agg_v, sem):
    wid = _worker_id()
    node_base = wid * NPT

    zero16 = jnp.zeros((16,), jnp.float32)

    def zrow(i, _):
        for k in range(D // 16):
            agg_v[i, pl.ds(k * 16, 16)] = zero16
        return 0

    lax.fori_loop(0, NPT + 1, zrow, 0)

    pltpu.sync_copy(est_hbm, ebuf)
    ev = ebuf[pl.ds(wid, 16)]
    e0 = ev[0]
    e1 = ev[1]
    a0 = (e0 // 8) * 8
    nchunks = (e1 - a0 + (K - 1)) // K

    def group_body(g, car):
        d_prev = car[0]
        acc = list(car[1:])
        dvec = dstb[pl.ds(g * 16, 16)] - node_base
        for lane in range(16):
            dl = dvec[lane]
            changed = dl != d_prev
            valid = jnp.logical_and(dl >= 0, dl < NPT)
            dlc = jnp.where(valid, dl, NPT)
            j = g * 16 + lane
            for k in range(D // 16):
                r = rows_v[j, pl.ds(k * 16, 16)]
                a = jnp.where(changed, r, jnp.maximum(acc[k], r))
                agg_v[dlc, pl.ds(k * 16, 16)] = a
                acc[k] = a
            d_prev = dl
        return (d_prev, *acc)

    def chunk_body(c, car):
        base_e = a0 + c * K
        pltpu.sync_copy(src_hbm.at[pl.ds(base_e, K)], idx_v)
        pltpu.sync_copy(dst_hbm.at[pl.ds(base_e, K)], dstb)
        pltpu.async_copy(h_hbm.at[idx_v], rows_v, sem).wait()
        return lax.fori_loop(0, K // 16, group_body, car)

    car0 = (jnp.int32(-(1 << 30)),) + tuple(
        jnp.zeros((16,), jnp.float32) for _ in range(D // 16))
    lax.fori_loop(0, nchunks, chunk_body, car0)

    pltpu.sync_copy(agg_v.at[pl.ds(0, NPT)], out_hbm.at[pl.ds(node_base, NPT)])


def _seg_max(h, src_pad, dst_pad, estarts):
    mesh = plsc.VectorSubcoreMesh(core_axis_name="c", subcore_axis_name="s",
                                  num_cores=NC, num_subcores=NS)
    f = pl.kernel(
        _seg_max_body,
        out_type=jax.ShapeDtypeStruct((NPAD, D), jnp.float32),
        mesh=mesh,
        scratch_types=[
            pltpu.VMEM((48,), jnp.int32),
            pltpu.VMEM((K,), jnp.int32),
            pltpu.VMEM((K,), jnp.int32),
            pltpu.VMEM((K, D), jnp.float32),
            pltpu.VMEM((NPT + 8, D), jnp.float32),
            pltpu.SemaphoreType.DMA,
        ],
    )
    return f(src_pad, dst_pad, estarts, h)


def _dense_body(agg_ref, h_ref, wl_ref, wr_ref, bl_ref, o_ref, *, relu):
    dn = (((1,), (1,)), ((), ()))
    o = (lax.dot_general(agg_ref[...], wl_ref[...], dn,
                         preferred_element_type=jnp.float32)
         + lax.dot_general(h_ref[...], wr_ref[...], dn,
                           preferred_element_type=jnp.float32)
         + bl_ref[...])
    if relu:
        o = jnp.maximum(o, 0.0)
    o_ref[...] = o


def _dense(agg, h, Wl, bl, Wr, relu):
    R = 2000
    grid = N // R
    return pl.pallas_call(
        functools.partial(_dense_body, relu=relu),
        grid=(grid,),
        in_specs=[
            pl.BlockSpec((R, D), lambda i: (i, 0)),
            pl.BlockSpec((R, D), lambda i: (i, 0)),
            pl.BlockSpec((D, D), lambda i: (0, 0)),
            pl.BlockSpec((D, D), lambda i: (0, 0)),
            pl.BlockSpec((1, D), lambda i: (0, 0)),
        ],
        out_specs=pl.BlockSpec((R, D), lambda i: (i, 0)),
        out_shape=jax.ShapeDtypeStruct((N, D), jnp.float32),
    )(agg, h, Wl, Wr, bl)


def kernel(x, edge_index, Wl1, bl1, Wr1, Wl2, bl2, Wr2, Wl3, bl3, Wr3,
           Wl4, bl4, Wr4, Wl5, bl5, Wr5, Wl6, bl6, Wr6):
    src, dst = edge_index[0], edge_index[1]
    dst_s, src_s = lax.sort((dst, src), num_keys=1)
    src_pad = jnp.concatenate([src_s, jnp.zeros((K,), jnp.int32)])
    dst_pad = jnp.concatenate([dst_s, jnp.full((K,), N, jnp.int32)])
    bounds = (jnp.arange(NW + 1, dtype=jnp.int32) * NPT)
    estarts = jnp.searchsorted(dst_s, bounds).astype(jnp.int32)
    estarts = jnp.concatenate(
        [estarts, jnp.zeros((48 - (NW + 1),), jnp.int32)])

    h = x
    layers = [(Wl1, bl1, Wr1, True), (Wl2, bl2, Wr2, True),
              (Wl3, bl3, Wr3, True), (Wl4, bl4, Wr4, True),
              (Wl5, bl5, Wr5, True), (Wl6, bl6, Wr6, False)]
    for Wl, bl, Wr, relu in layers:
        agg = _seg_max(h, src_pad, dst_pad, estarts)[:N]
        h = _dense(agg, h, Wl, bl.reshape(1, D), Wr, relu)
    return h


# speedup vs baseline: 2.9317x; 1.2746x over previous
import functools

import jax
import jax.numpy as jnp
from jax import lax
from jax.experimental import pallas as pl
from jax.experimental.pallas import tpu as pltpu
from jax.experimental.pallas import tpu_sc as plsc

N = 10000
E = 320000
D = 128

NC = 2
NS = 16
NW = NC * NS
NPT = 320
NPAD = NW * NPT
K = 256
PAD = 4 * K


def _worker_id():
    return lax.axis_index("s") * NC + lax.axis_index("c")


def _seg_max_body(src_hbm, dst_hbm, est_hbm, h_hbm, out_hbm,
                  ebuf, idx0, idx1, dst0, dst1, rows0, rows1, agg_v,
                  sg0, sg1, si0, si1, sd0, sd1):
    idx_v = (idx0, idx1)
    dstb = (dst0, dst1)
    rows_v = (rows0, rows1)
    sg = (sg0, sg1)
    si = (si0, si1)
    sd = (sd0, sd1)

    wid = _worker_id()
    node_base = wid * NPT

    zero16 = jnp.zeros((16,), jnp.float32)

    def zrow(i, _):
        for k in range(D // 16):
            agg_v[i, pl.ds(k * 16, 16)] = zero16
        return 0

    lax.fori_loop(0, NPT + 1, zrow, 0)

    pltpu.sync_copy(est_hbm, ebuf)
    ev = ebuf[pl.ds(wid, 16)]
    e0 = ev[0]
    e1 = ev[1]
    a0 = (e0 // 8) * 8
    nchunks = (e1 - a0 + (K - 1)) // K
    npairs = jnp.maximum((nchunks + 1) // 2, 1)

    def issue_load(p, base_e):
        pltpu.make_async_copy(
            src_hbm.at[pl.ds(base_e, K)], idx_v[p], si[p]).start()
        pltpu.make_async_copy(
            dst_hbm.at[pl.ds(base_e, K)], dstb[p], sd[p]).start()

    def wait_load(p):
        pltpu.make_async_copy(
            src_hbm.at[pl.ds(0, K)], idx_v[p], si[p]).wait()
        pltpu.make_async_copy(
            dst_hbm.at[pl.ds(0, K)], dstb[p], sd[p]).wait()

    def issue_gather(p):
        pltpu.make_async_copy(h_hbm.at[idx_v[p]], rows_v[p], sg[p]).start()

    def wait_gather(p):
        pltpu.make_async_copy(h_hbm.at[idx_v[p]], rows_v[p], sg[p]).wait()

    def compute(p, car):
        def group_body(g, gcar):
            d_prev = gcar[0]
            acc = list(gcar[1:])
            dvec = dstb[p][pl.ds(g * 16, 16)] - node_base
            for lane in range(16):
                dl = dvec[lane]
                changed = dl != d_prev
                valid = jnp.logical_and(dl >= 0, dl < NPT)
                dlc = jnp.where(valid, dl, NPT)
                j = g * 16 + lane
                for k in range(D // 16):
                    r = rows_v[p][j, pl.ds(k * 16, 16)]
                    a = jnp.where(changed, r, jnp.maximum(acc[k], r))
                    agg_v[dlc, pl.ds(k * 16, 16)] = a
                    acc[k] = a
                d_prev = dl
            return (d_prev, *acc)

        return lax.fori_loop(0, K // 16, group_body, car)

    issue_load(0, a0)
    wait_load(0)
    issue_gather(0)
    issue_load(1, a0 + K)

    def pair_body(i, car):
        for b in (0, 1):
            c = 2 * i + b
            p = b
            q = 1 - b
            wait_load(q)
            issue_gather(q)
            wait_gather(p)
            car = compute(p, car)
            issue_load(p, a0 + (c + 2) * K)
        return car

    car0 = (jnp.int32(-(1 << 30)),) + tuple(
        jnp.zeros((16,), jnp.float32) for _ in range(D // 16))
    lax.fori_loop(0, npairs, pair_body, car0)

    wait_gather(0)
    wait_load(1)

    pltpu.sync_copy(agg_v.at[pl.ds(0, NPT)], out_hbm.at[pl.ds(node_base, NPT)])


def _seg_max(h, src_pad, dst_pad, estarts):
    mesh = plsc.VectorSubcoreMesh(core_axis_name="c", subcore_axis_name="s",
                                  num_cores=NC, num_subcores=NS)
    f = pl.kernel(
        _seg_max_body,
        out_type=jax.ShapeDtypeStruct((NPAD, D), jnp.float32),
        mesh=mesh,
        scratch_types=[
            pltpu.VMEM((48,), jnp.int32),
            pltpu.VMEM((K,), jnp.int32),
            pltpu.VMEM((K,), jnp.int32),
            pltpu.VMEM((K,), jnp.int32),
            pltpu.VMEM((K,), jnp.int32),
            pltpu.VMEM((K, D), jnp.float32),
            pltpu.VMEM((K, D), jnp.float32),
            pltpu.VMEM((NPT + 8, D), jnp.float32),
            pltpu.SemaphoreType.DMA,
            pltpu.SemaphoreType.DMA,
            pltpu.SemaphoreType.DMA,
            pltpu.SemaphoreType.DMA,
            pltpu.SemaphoreType.DMA,
            pltpu.SemaphoreType.DMA,
        ],
    )
    return f(src_pad, dst_pad, estarts, h)


def _dense_body(agg_ref, h_ref, wl_ref, wr_ref, bl_ref, o_ref, *, relu):
    dn = (((1,), (1,)), ((), ()))
    o = (lax.dot_general(agg_ref[...], wl_ref[...], dn,
                         preferred_element_type=jnp.float32)
         + lax.dot_general(h_ref[...], wr_ref[...], dn,
                           preferred_element_type=jnp.float32)
         + bl_ref[...])
    if relu:
        o = jnp.maximum(o, 0.0)
    o_ref[...] = o


def _dense(agg, h, Wl, bl, Wr, relu):
    R = 2000
    grid = N // R
    return pl.pallas_call(
        functools.partial(_dense_body, relu=relu),
        grid=(grid,),
        in_specs=[
            pl.BlockSpec((R, D), lambda i: (i, 0)),
            pl.BlockSpec((R, D), lambda i: (i, 0)),
            pl.BlockSpec((D, D), lambda i: (0, 0)),
            pl.BlockSpec((D, D), lambda i: (0, 0)),
            pl.BlockSpec((1, D), lambda i: (0, 0)),
        ],
        out_specs=pl.BlockSpec((R, D), lambda i: (i, 0)),
        out_shape=jax.ShapeDtypeStruct((N, D), jnp.float32),
    )(agg, h, Wl, Wr, bl)


def kernel(x, edge_index, Wl1, bl1, Wr1, Wl2, bl2, Wr2, Wl3, bl3, Wr3,
           Wl4, bl4, Wr4, Wl5, bl5, Wr5, Wl6, bl6, Wr6):
    src, dst = edge_index[0], edge_index[1]
    dst_s, src_s = lax.sort((dst, src), num_keys=1)
    src_pad = jnp.concatenate([src_s, jnp.zeros((PAD,), jnp.int32)])
    dst_pad = jnp.concatenate([dst_s, jnp.full((PAD,), N, jnp.int32)])
    bounds = (jnp.arange(NW + 1, dtype=jnp.int32) * NPT)
    estarts = jnp.searchsorted(dst_s, bounds).astype(jnp.int32)
    estarts = jnp.concatenate(
        [estarts, jnp.zeros((48 - (NW + 1),), jnp.int32)])

    h = x
    layers = [(Wl1, bl1, Wr1, True), (Wl2, bl2, Wr2, True),
              (Wl3, bl3, Wr3, True), (Wl4, bl4, Wr4, True),
              (Wl5, bl5, Wr5, True), (Wl6, bl6, Wr6, False)]
    for Wl, bl, Wr, relu in layers:
        agg = _seg_max(h, src_pad, dst_pad, estarts)[:N]
        h = _dense(agg, h, Wl, bl.reshape(1, D), Wr, relu)
    return h


# speedup vs baseline: 6.5827x; 2.2453x over previous
import functools

import jax
import jax.numpy as jnp
from jax import lax
from jax.experimental import pallas as pl
from jax.experimental.pallas import tpu as pltpu
from jax.experimental.pallas import tpu_sc as plsc

N = 10000
E = 320000
D = 128

NC = 2
NS = 16
NW = NC * NS
NPT = 320
NPAD = NW * NPT
K = 256
PAD = 4 * K


def _worker_id():
    return lax.axis_index("s") * NC + lax.axis_index("c")


def _seg_max_body(src_hbm, dst_hbm, est_hbm, h_hbm, out_hbm,
                  ebuf, idx0, idx1, dst0, dst1, rows0, rows1, agg_v,
                  sg0, sg1, si0, si1, sd0, sd1):
    idx_v = (idx0, idx1)
    dstb = (dst0, dst1)
    rows_v = (rows0, rows1)
    sg = (sg0, sg1)
    si = (si0, si1)
    sd = (sd0, sd1)

    wid = _worker_id()
    node_base = wid * NPT

    zero16 = jnp.zeros((16,), jnp.float32)

    def zrow(i, _):
        for k in range(D // 16):
            agg_v[pl.ds(i * D + k * 16, 16)] = zero16
        return 0

    lax.fori_loop(0, NPT + 1, zrow, 0)

    pltpu.sync_copy(est_hbm, ebuf)
    ev = ebuf[pl.ds(wid, 16)]
    e0 = ev[0]
    e1 = ev[1]
    a0 = (e0 // 8) * 8
    nchunks = (e1 - a0 + (K - 1)) // K
    npairs = jnp.maximum((nchunks + 1) // 2, 1)

    def issue_load(p, base_e):
        pltpu.make_async_copy(
            src_hbm.at[pl.ds(base_e, K)], idx_v[p], si[p]).start()
        pltpu.make_async_copy(
            dst_hbm.at[pl.ds(base_e, K)], dstb[p], sd[p]).start()

    def wait_load(p):
        pltpu.make_async_copy(
            src_hbm.at[pl.ds(0, K)], idx_v[p], si[p]).wait()
        pltpu.make_async_copy(
            dst_hbm.at[pl.ds(0, K)], dstb[p], sd[p]).wait()

    def issue_gather(p):
        pltpu.make_async_copy(h_hbm.at[idx_v[p]], rows_v[p], sg[p]).start()

    def wait_gather(p):
        pltpu.make_async_copy(h_hbm.at[idx_v[p]], rows_v[p], sg[p]).wait()

    def flush(d_prev, acc):
        valid = jnp.logical_and(d_prev >= 0, d_prev < NPT)
        dlp = jnp.where(valid, d_prev, NPT)
        for k in range(D // 16):
            agg_v[pl.ds(dlp * D + k * 16, 16)] = acc[k]

    def compute(p, car):
        def group_body(g, gcar):
            d_prev = gcar[0]
            acc = list(gcar[1:])
            dvec = dstb[p][pl.ds(g * 16, 16)] - node_base
            for lane in range(16):
                dl = dvec[lane]
                changed = dl != d_prev

                @pl.when(changed)
                def _(d_prev=d_prev, acc=tuple(acc)):
                    flush(d_prev, acc)

                j = g * 16 + lane
                for k in range(D // 16):
                    r = rows_v[p][j, pl.ds(k * 16, 16)]
                    acc[k] = jnp.where(changed, r, jnp.maximum(acc[k], r))
                d_prev = dl
            return (d_prev, *acc)

        return lax.fori_loop(0, K // 16, group_body, car)

    issue_load(0, a0)
    wait_load(0)
    issue_gather(0)
    issue_load(1, a0 + K)

    def pair_body(i, car):
        for b in (0, 1):
            c = 2 * i + b
            p = b
            q = 1 - b
            wait_load(q)
            issue_gather(q)
            wait_gather(p)
            car = compute(p, car)
            issue_load(p, a0 + (c + 2) * K)
        return car

    car0 = (jnp.int32(-(1 << 30)),) + tuple(
        jnp.zeros((16,), jnp.float32) for _ in range(D // 16))
    car = lax.fori_loop(0, npairs, pair_body, car0)

    wait_gather(0)
    wait_load(1)

    flush(car[0], list(car[1:]))

    pltpu.sync_copy(agg_v.at[pl.ds(0, NPT * D)],
                    out_hbm.at[pl.ds(node_base * D, NPT * D)])


def _seg_max(h, src_pad, dst_pad, estarts):
    mesh = plsc.VectorSubcoreMesh(core_axis_name="c", subcore_axis_name="s",
                                  num_cores=NC, num_subcores=NS)
    f = pl.kernel(
        _seg_max_body,
        out_type=jax.ShapeDtypeStruct((NPAD * D,), jnp.float32),
        mesh=mesh,
        scratch_types=[
            pltpu.VMEM((48,), jnp.int32),
            pltpu.VMEM((K,), jnp.int32),
            pltpu.VMEM((K,), jnp.int32),
            pltpu.VMEM((K,), jnp.int32),
            pltpu.VMEM((K,), jnp.int32),
            pltpu.VMEM((K, D), jnp.float32),
            pltpu.VMEM((K, D), jnp.float32),
            pltpu.VMEM(((NPT + 1) * D,), jnp.float32),
            pltpu.SemaphoreType.DMA,
            pltpu.SemaphoreType.DMA,
            pltpu.SemaphoreType.DMA,
            pltpu.SemaphoreType.DMA,
            pltpu.SemaphoreType.DMA,
            pltpu.SemaphoreType.DMA,
        ],
    )
    return f(src_pad, dst_pad, estarts, h).reshape(NPAD, D)


def _dense_body(agg_ref, h_ref, wl_ref, wr_ref, bl_ref, o_ref, *, relu):
    dn = (((1,), (1,)), ((), ()))
    o = (lax.dot_general(agg_ref[...], wl_ref[...], dn,
                         preferred_element_type=jnp.float32)
         + lax.dot_general(h_ref[...], wr_ref[...], dn,
                           preferred_element_type=jnp.float32)
         + bl_ref[...])
    if relu:
        o = jnp.maximum(o, 0.0)
    o_ref[...] = o


def _dense(agg, h, Wl, bl, Wr, relu):
    R = 2000
    grid = N // R
    return pl.pallas_call(
        functools.partial(_dense_body, relu=relu),
        grid=(grid,),
        in_specs=[
            pl.BlockSpec((R, D), lambda i: (i, 0)),
            pl.BlockSpec((R, D), lambda i: (i, 0)),
            pl.BlockSpec((D, D), lambda i: (0, 0)),
            pl.BlockSpec((D, D), lambda i: (0, 0)),
            pl.BlockSpec((1, D), lambda i: (0, 0)),
        ],
        out_specs=pl.BlockSpec((R, D), lambda i: (i, 0)),
        out_shape=jax.ShapeDtypeStruct((N, D), jnp.float32),
    )(agg, h, Wl, Wr, bl)


def kernel(x, edge_index, Wl1, bl1, Wr1, Wl2, bl2, Wr2, Wl3, bl3, Wr3,
           Wl4, bl4, Wr4, Wl5, bl5, Wr5, Wl6, bl6, Wr6):
    src, dst = edge_index[0], edge_index[1]
    dst_s, src_s = lax.sort((dst, src), num_keys=1)
    src_pad = jnp.concatenate([src_s, jnp.zeros((PAD,), jnp.int32)])
    dst_pad = jnp.concatenate([dst_s, jnp.full((PAD,), N, jnp.int32)])
    bounds = (jnp.arange(NW + 1, dtype=jnp.int32) * NPT)
    estarts = jnp.searchsorted(dst_s, bounds).astype(jnp.int32)
    estarts = jnp.concatenate(
        [estarts, jnp.zeros((48 - (NW + 1),), jnp.int32)])

    h = x
    layers = [(Wl1, bl1, Wr1, True), (Wl2, bl2, Wr2, True),
              (Wl3, bl3, Wr3, True), (Wl4, bl4, Wr4, True),
              (Wl5, bl5, Wr5, True), (Wl6, bl6, Wr6, False)]
    for Wl, bl, Wr, relu in layers:
        agg = _seg_max(h, src_pad, dst_pad, estarts)[:N]
        h = _dense(agg, h, Wl, bl.reshape(1, D), Wr, relu)
    return h


# speedup vs baseline: 6.8406x; 1.0392x over previous
import functools

import jax
import jax.numpy as jnp
from jax import lax
from jax.experimental import pallas as pl
from jax.experimental.pallas import tpu as pltpu
from jax.experimental.pallas import tpu_sc as plsc

N = 10000
E = 320000
D = 128

NC = 2
NS = 16
NW = NC * NS
NPT = 320
NPAD = NW * NPT
K = 256
PAD = 4 * K


def _worker_id():
    return lax.axis_index("s") * NC + lax.axis_index("c")


def _seg_max_body(src_hbm, dst_hbm, est_hbm, h_hbm, out_hbm,
                  ebuf, idx0, idx1, dst0, dst1, rows0, rows1, agg_v,
                  sg0, sg1, si0, si1, sd0, sd1):
    idx_v = (idx0, idx1)
    dstb = (dst0, dst1)
    rows_v = (rows0, rows1)
    sg = (sg0, sg1)
    si = (si0, si1)
    sd = (sd0, sd1)

    wid = _worker_id()
    node_base = wid * NPT

    zero16 = jnp.zeros((16,), jnp.float32)

    def zrow(i, _):
        for k in range(D // 16):
            agg_v[pl.ds(i * D + k * 16, 16)] = zero16
        return 0

    lax.fori_loop(0, NPT + 1, zrow, 0)

    pltpu.sync_copy(est_hbm, ebuf)
    ev = ebuf[pl.ds(wid, 16)]
    e0 = ev[0]
    e1 = ev[1]
    a0 = (e0 // 8) * 8
    nchunks = (e1 - a0 + (K - 1)) // K
    npairs = jnp.maximum((nchunks + 1) // 2, 1)

    def issue_load(p, base_e):
        pltpu.make_async_copy(
            src_hbm.at[pl.ds(base_e, K)], idx_v[p], si[p]).start()
        pltpu.make_async_copy(
            dst_hbm.at[pl.ds(base_e, K)], dstb[p], sd[p]).start()

    def wait_load(p):
        pltpu.make_async_copy(
            src_hbm.at[pl.ds(0, K)], idx_v[p], si[p]).wait()
        pltpu.make_async_copy(
            dst_hbm.at[pl.ds(0, K)], dstb[p], sd[p]).wait()

    def issue_gather(p):
        pltpu.make_async_copy(h_hbm.at[idx_v[p]], rows_v[p], sg[p]).start()

    def wait_gather(p):
        pltpu.make_async_copy(h_hbm.at[idx_v[p]], rows_v[p], sg[p]).wait()

    def flush(d_prev, acc):
        valid = jnp.logical_and(d_prev >= 0, d_prev < NPT)
        dlp = jnp.where(valid, d_prev, NPT)
        for k in range(D // 16):
            agg_v[pl.ds(dlp * D + k * 16, 16)] = acc[k]

    def compute(p, car):
        def group_body(g, gcar):
            d_prev = gcar[0]
            acc = list(gcar[1:])
            dvec = dstb[p][pl.ds(g * 16, 16)] - node_base
            for lane in range(16):
                dl = dvec[lane]
                changed = dl != d_prev

                @pl.when(changed)
                def _(d_prev=d_prev, acc=tuple(acc)):
                    flush(d_prev, acc)

                j = g * 16 + lane
                for k in range(D // 16):
                    r = rows_v[p][j, pl.ds(k * 16, 16)]
                    acc[k] = jnp.where(changed, r, jnp.maximum(acc[k], r))
                d_prev = dl
            return (d_prev, *acc)

        return lax.fori_loop(0, K // 16, group_body, car)

    issue_load(0, a0)
    wait_load(0)
    issue_gather(0)
    issue_load(1, a0 + K)

    def pair_body(i, car):
        for b in (0, 1):
            c = 2 * i + b
            p = b
            q = 1 - b
            wait_load(q)
            issue_gather(q)
            wait_gather(p)
            car = compute(p, car)
            issue_load(p, a0 + (c + 2) * K)
        return car

    car0 = (jnp.int32(-(1 << 30)),) + tuple(
        jnp.zeros((16,), jnp.float32) for _ in range(D // 16))
    car = lax.fori_loop(0, npairs, pair_body, car0)

    wait_gather(0)
    wait_load(1)

    flush(car[0], list(car[1:]))

    pltpu.sync_copy(agg_v.at[pl.ds(0, NPT * D)],
                    out_hbm.at[pl.ds(node_base * D, NPT * D)])


def _seg_max(h, src_pad, dst_pad, estarts):
    mesh = plsc.VectorSubcoreMesh(core_axis_name="c", subcore_axis_name="s",
                                  num_cores=NC, num_subcores=NS)
    f = pl.kernel(
        _seg_max_body,
        out_type=jax.ShapeDtypeStruct((NPAD * D,), jnp.float32),
        mesh=mesh,
        scratch_types=[
            pltpu.VMEM((48,), jnp.int32),
            pltpu.VMEM((K,), jnp.int32),
            pltpu.VMEM((K,), jnp.int32),
            pltpu.VMEM((K,), jnp.int32),
            pltpu.VMEM((K,), jnp.int32),
            pltpu.VMEM((K, D), jnp.float32),
            pltpu.VMEM((K, D), jnp.float32),
            pltpu.VMEM(((NPT + 1) * D,), jnp.float32),
            pltpu.SemaphoreType.DMA,
            pltpu.SemaphoreType.DMA,
            pltpu.SemaphoreType.DMA,
            pltpu.SemaphoreType.DMA,
            pltpu.SemaphoreType.DMA,
            pltpu.SemaphoreType.DMA,
        ],
    )
    return f(src_pad, dst_pad, estarts, h).reshape(NPAD, D)


def _dense_body(agg_ref, h_ref, wl_ref, wr_ref, bl_ref, o_ref, *, relu):
    dn = (((1,), (1,)), ((), ()))
    o = (lax.dot_general(agg_ref[...], wl_ref[...], dn,
                         preferred_element_type=jnp.float32)
         + lax.dot_general(h_ref[...], wr_ref[...], dn,
                           preferred_element_type=jnp.float32)
         + bl_ref[...])
    if relu:
        o = jnp.maximum(o, 0.0)
    o_ref[...] = o


def _dense(agg, h, Wl, bl, Wr, relu):
    R = 2000
    grid = N // R
    return pl.pallas_call(
        functools.partial(_dense_body, relu=relu),
        grid=(grid,),
        in_specs=[
            pl.BlockSpec((R, D), lambda i: (i, 0)),
            pl.BlockSpec((R, D), lambda i: (i, 0)),
            pl.BlockSpec((D, D), lambda i: (0, 0)),
            pl.BlockSpec((D, D), lambda i: (0, 0)),
            pl.BlockSpec((1, D), lambda i: (0, 0)),
        ],
        out_specs=pl.BlockSpec((R, D), lambda i: (i, 0)),
        out_shape=jax.ShapeDtypeStruct((N, D), jnp.float32),
    )(agg, h, Wl, Wr, bl)


def kernel(x, edge_index, Wl1, bl1, Wr1, Wl2, bl2, Wr2, Wl3, bl3, Wr3,
           Wl4, bl4, Wr4, Wl5, bl5, Wr5, Wl6, bl6, Wr6):
    src, dst = edge_index[0], edge_index[1]
    packed = lax.sort((dst << 14) | src)
    dst_s = packed >> 14
    src_s = packed & 16383
    src_pad = jnp.concatenate([src_s, jnp.zeros((PAD,), jnp.int32)])
    dst_pad = jnp.concatenate([dst_s, jnp.full((PAD,), N, jnp.int32)])
    bounds = (jnp.arange(NW + 1, dtype=jnp.int32) * NPT) << 14
    estarts = jnp.searchsorted(packed, bounds).astype(jnp.int32)
    estarts = jnp.concatenate(
        [estarts, jnp.zeros((48 - (NW + 1),), jnp.int32)])

    h = x
    layers = [(Wl1, bl1, Wr1, True), (Wl2, bl2, Wr2, True),
              (Wl3, bl3, Wr3, True), (Wl4, bl4, Wr4, True),
              (Wl5, bl5, Wr5, True), (Wl6, bl6, Wr6, False)]
    for Wl, bl, Wr, relu in layers:
        agg = _seg_max(h, src_pad, dst_pad, estarts)
        h = _dense(agg, h, Wl, bl.reshape(1, D), Wr, relu)
    return h
